# baseline (speedup 1.0000x reference)
import functools

import jax
import jax.numpy as jnp
from jax import lax
from jax.experimental import pallas as pl
from jax.experimental.pallas import tpu as pltpu
from jax.experimental.pallas import tpu_sc as plsc

F = 26
V = 100000
D = 32
B = 16384

NW = 32
PLANES = F * D
PPW = PLANES // NW
OUTC = 4096


CV = 16384
NVB = pl.cdiv(V + 1, CV)


def _scale_body(w_ref, o_ref):
    iv = pl.program_id(1)

    @pl.when(iv == 0)
    def _init():
        o_ref[...] = jnp.zeros_like(o_ref)

    vals = jnp.abs(w_ref[0])

    @pl.when(iv < NVB - 1)
    def _body():
        o_ref[...] = jnp.maximum(o_ref[...], jnp.max(vals))

    @pl.when(iv == NVB - 1)
    def _finish():
        lim = (V + 1) - iv * CV
        col = lax.broadcasted_iota(jnp.int32, (D, CV), 1)
        m = jnp.max(jnp.where(col < lim, vals, 0.0))
        o_ref[...] = 0.2 / jnp.maximum(o_ref[...], m)


def _tc_scale(wt):
    return pl.pallas_call(
        _scale_body,
        grid=(F, NVB),
        in_specs=[pl.BlockSpec((1, D, CV), lambda f, v: (f, 0, v))],
        out_specs=pl.BlockSpec((1, 8, 128), lambda f, v: (f, 0, 0)),
        out_shape=jax.ShapeDtypeStruct((F, 8, 128), jnp.float32),
    )(wt)


CB = 16384


def _tanh_body(g_ref, s_ref, o_ref):
    o_ref[...] = jnp.tanh(g_ref[...] * s_ref[0, 0, 0])


def _tc_tanh(raw, scale_full):
    return pl.pallas_call(
        _tanh_body,
        grid=(F, B // CB),
        in_specs=[
            pl.BlockSpec((1, D, CB), lambda f, b: (f, 0, b)),
            pl.BlockSpec((1, 8, 128), lambda f, b: (f, 0, 0)),
        ],
        out_specs=pl.BlockSpec((1, D, CB), lambda f, b: (f, 0, b)),
        out_shape=jax.ShapeDtypeStruct((F, D, B), jnp.float32),
    )(raw, scale_full)


_mesh = plsc.VectorSubcoreMesh(
    core_axis_name="c", subcore_axis_name="s", num_cores=2, num_subcores=16
)


@functools.partial(
    pl.kernel,
    mesh=_mesh,
    compiler_params=pltpu.CompilerParams(needs_layout_passes=False),
    out_type=jax.ShapeDtypeStruct((F, D, B), jnp.float32),
    scratch_types=[
        pltpu.VMEM((V + 1,), jnp.float32),
        pltpu.VMEM((B,), jnp.int32),
        pltpu.VMEM((2, OUTC), jnp.float32),
        pltpu.SemaphoreType.DMA,
        pltpu.SemaphoreType.DMA,
        pltpu.SemaphoreType.DMA,
    ],
)
def _sc_gather(wt_hbm, xt_hbm, out_hbm, plane_v, x_v, o_v, sem, osem0, osem1):
    wid = lax.axis_index("s") * 2 + lax.axis_index("c")
    p0 = wid * PPW

    f0 = lax.shift_right_logical(p0, 5)
    pltpu.sync_copy(xt_hbm.at[f0, :], x_v)

    def plane_body(pi, carry):
        p = p0 + pi
        f = lax.shift_right_logical(p, 5)
        d = lax.bitwise_and(p, 31)

        plane_cp = pltpu.async_copy(wt_hbm.at[f, d, :], plane_v, sem)

        @pl.when(jnp.logical_and(d == 0, pi > 0))
        def _new_field():
            pltpu.sync_copy(xt_hbm.at[f, :], x_v)

        plane_cp.wait()

        for q in range(4):
            slot = q % 2
            osem = osem0 if slot == 0 else osem1

            def _drain():
                pltpu.make_async_copy(
                    out_hbm.at[f, d, pl.ds(q * OUTC, OUTC)],
                    o_v.at[slot], osem,
                ).wait()

            if q >= 2:
                _drain()
            else:
                pl.when(pi > 0)(_drain)

            @plsc.parallel_loop(0, OUTC, 16, unroll=8)
            def _chunk_loop(i):
                xi = x_v[pl.ds(q * OUTC + i, 16)] + 1
                o_v[slot, pl.ds(i, 16)] = plsc.load_gather(plane_v, [xi])

            pltpu.async_copy(
                o_v.at[slot], out_hbm.at[f, d, pl.ds(q * OUTC, OUTC)], osem
            )
        return carry

    lax.fori_loop(0, PPW, plane_body, 0)

    pltpu.make_async_copy(
        out_hbm.at[0, 0, pl.ds(0, OUTC)], o_v.at[0], osem0).wait()
    pltpu.make_async_copy(
        out_hbm.at[0, 0, pl.ds(0, OUTC)], o_v.at[1], osem1).wait()


def kernel(x, W):
    wt = jnp.transpose(W, (0, 2, 1))
    xt = jnp.transpose(x, (1, 0))
    raw = _sc_gather(wt, xt)
    scale_full = _tc_scale(wt)
    out_t = _tc_tanh(raw, scale_full)
    return jnp.transpose(out_t, (2, 0, 1))


# speedup vs baseline: 1.0517x; 1.0517x over previous
import functools

import jax
import jax.numpy as jnp
from jax import lax
from jax.experimental import pallas as pl
from jax.experimental.pallas import tpu as pltpu
from jax.experimental.pallas import tpu_sc as plsc

F = 26
V = 100000
D = 32
B = 16384

NW = 32
PLANES = F * D
PPW = PLANES // NW
OUTC = 4096


CV = 50048
NVB = pl.cdiv(V + 1, CV)


def _scale_body(w_ref, o_ref):
    iv = pl.program_id(1)

    @pl.when(iv == 0)
    def _init():
        o_ref[...] = jnp.zeros_like(o_ref)

    vals = jnp.abs(w_ref[0])

    @pl.when(iv < NVB - 1)
    def _body():
        o_ref[...] = jnp.maximum(o_ref[...], jnp.max(vals))

    @pl.when(iv == NVB - 1)
    def _finish():
        lim = (V + 1) - iv * CV
        col = lax.broadcasted_iota(jnp.int32, (D, CV), 1)
        m = jnp.max(jnp.where(col < lim, vals, 0.0))
        o_ref[...] = 0.2 / jnp.maximum(o_ref[...], m)


def _tc_scale(wt):
    return pl.pallas_call(
        _scale_body,
        grid=(F, NVB),
        in_specs=[pl.BlockSpec((1, D, CV), lambda f, v: (f, 0, v))],
        out_specs=pl.BlockSpec((1, 8, 128), lambda f, v: (f, 0, 0)),
        out_shape=jax.ShapeDtypeStruct((F, 8, 128), jnp.float32),
    )(wt)


CB = 16384


def _tanh_body(g_ref, s_ref, o_ref):
    o_ref[...] = jnp.tanh(g_ref[...] * s_ref[0, 0, 0])


def _tc_tanh(raw, scale_full):
    return pl.pallas_call(
        _tanh_body,
        grid=(F, B // CB),
        in_specs=[
            pl.BlockSpec((1, D, CB), lambda f, b: (f, 0, b)),
            pl.BlockSpec((1, 8, 128), lambda f, b: (f, 0, 0)),
        ],
        out_specs=pl.BlockSpec((1, D, CB), lambda f, b: (f, 0, b)),
        out_shape=jax.ShapeDtypeStruct((F, D, B), jnp.float32),
    )(raw, scale_full)


_mesh = plsc.VectorSubcoreMesh(
    core_axis_name="c", subcore_axis_name="s", num_cores=2, num_subcores=16
)


@functools.partial(
    pl.kernel,
    mesh=_mesh,
    compiler_params=pltpu.CompilerParams(needs_layout_passes=False),
    out_type=jax.ShapeDtypeStruct((F, D, B), jnp.float32),
    scratch_types=[
        pltpu.VMEM((V + 1,), jnp.float32),
        pltpu.VMEM((B,), jnp.int32),
        pltpu.VMEM((2, OUTC), jnp.float32),
        pltpu.SemaphoreType.DMA,
        pltpu.SemaphoreType.DMA,
        pltpu.SemaphoreType.DMA,
    ],
)
def _sc_gather(wt_hbm, xt_hbm, out_hbm, plane_v, x_v, o_v, sem, osem0, osem1):
    wid = lax.axis_index("s") * 2 + lax.axis_index("c")
    p0 = wid * PPW

    f0 = lax.shift_right_logical(p0, 5)
    pltpu.sync_copy(xt_hbm.at[f0, :], x_v)

    def plane_body(pi, carry):
        p = p0 + pi
        f = lax.shift_right_logical(p, 5)
        d = lax.bitwise_and(p, 31)

        plane_cp = pltpu.async_copy(wt_hbm.at[f, d, :], plane_v, sem)

        @pl.when(jnp.logical_and(d == 0, pi > 0))
        def _new_field():
            pltpu.sync_copy(xt_hbm.at[f, :], x_v)

        plane_cp.wait()

        for q in range(4):
            slot = q % 2
            osem = osem0 if slot == 0 else osem1

            def _drain():
                pltpu.make_async_copy(
                    out_hbm.at[f, d, pl.ds(q * OUTC, OUTC)],
                    o_v.at[slot], osem,
                ).wait()

            if q >= 2:
                _drain()
            else:
                pl.when(pi > 0)(_drain)

            @plsc.parallel_loop(0, OUTC, 16, unroll=8)
            def _chunk_loop(i):
                xi = x_v[pl.ds(q * OUTC + i, 16)] + 1
                o_v[slot, pl.ds(i, 16)] = plsc.load_gather(plane_v, [xi])

            pltpu.async_copy(
                o_v.at[slot], out_hbm.at[f, d, pl.ds(q * OUTC, OUTC)], osem
            )
        return carry

    lax.fori_loop(0, PPW, plane_body, 0)

    pltpu.make_async_copy(
        out_hbm.at[0, 0, pl.ds(0, OUTC)], o_v.at[0], osem0).wait()
    pltpu.make_async_copy(
        out_hbm.at[0, 0, pl.ds(0, OUTC)], o_v.at[1], osem1).wait()


def kernel(x, W):
    wt = jnp.transpose(W, (0, 2, 1))
    xt = jnp.transpose(x, (1, 0))
    raw = _sc_gather(wt, xt)
    scale_full = _tc_scale(wt)
    out_t = _tc_tanh(raw, scale_full)
    return jnp.transpose(out_t, (2, 0, 1))


# speedup vs baseline: 1.0569x; 1.0049x over previous
import functools

import jax
import jax.numpy as jnp
from jax import lax
from jax.experimental import pallas as pl
from jax.experimental.pallas import tpu as pltpu
from jax.experimental.pallas import tpu_sc as plsc

F = 26
V = 100000
D = 32
B = 16384

NW = 32
PLANES = F * D
PPW = PLANES // NW
OUTC = 4096


CV = 32768
NVB = pl.cdiv(V + 1, CV)


def _scale_body(w_ref, o_ref):
    iv = pl.program_id(1)

    @pl.when(iv == 0)
    def _init():
        o_ref[...] = jnp.zeros_like(o_ref)

    vals = jnp.abs(w_ref[0])

    @pl.when(iv < NVB - 1)
    def _body():
        o_ref[...] = jnp.maximum(o_ref[...], jnp.max(vals))

    @pl.when(iv == NVB - 1)
    def _finish():
        lim = (V + 1) - iv * CV
        col = lax.broadcasted_iota(jnp.int32, (D, CV), 1)
        m = jnp.max(jnp.where(col < lim, vals, 0.0))
        o_ref[...] = 0.2 / jnp.maximum(o_ref[...], m)


def _tc_scale(wt):
    return pl.pallas_call(
        _scale_body,
        grid=(F, NVB),
        in_specs=[pl.BlockSpec((1, D, CV), lambda f, v: (f, 0, v))],
        out_specs=pl.BlockSpec((1, 8, 128), lambda f, v: (f, 0, 0)),
        out_shape=jax.ShapeDtypeStruct((F, 8, 128), jnp.float32),
    )(wt)


CB = 16384


def _tanh_body(g_ref, s_ref, o_ref):
    o_ref[...] = jnp.tanh(g_ref[...] * s_ref[0, 0, 0])


def _tc_tanh(raw, scale_full):
    return pl.pallas_call(
        _tanh_body,
        grid=(F, B // CB),
        in_specs=[
            pl.BlockSpec((1, D, CB), lambda f, b: (f, 0, b)),
            pl.BlockSpec((1, 8, 128), lambda f, b: (f, 0, 0)),
        ],
        out_specs=pl.BlockSpec((1, D, CB), lambda f, b: (f, 0, b)),
        out_shape=jax.ShapeDtypeStruct((F, D, B), jnp.float32),
    )(raw, scale_full)


_mesh = plsc.VectorSubcoreMesh(
    core_axis_name="c", subcore_axis_name="s", num_cores=2, num_subcores=16
)


@functools.partial(
    pl.kernel,
    mesh=_mesh,
    compiler_params=pltpu.CompilerParams(needs_layout_passes=False),
    out_type=jax.ShapeDtypeStruct((F, D, B), jnp.float32),
    scratch_types=[
        pltpu.VMEM((V + 1,), jnp.float32),
        pltpu.VMEM((B,), jnp.int32),
        pltpu.VMEM((2, OUTC), jnp.float32),
        pltpu.SemaphoreType.DMA,
        pltpu.SemaphoreType.DMA,
        pltpu.SemaphoreType.DMA,
    ],
)
def _sc_gather(wt_hbm, xt_hbm, out_hbm, plane_v, x_v, o_v, sem, osem0, osem1):
    wid = lax.axis_index("s") * 2 + lax.axis_index("c")
    p0 = wid * PPW

    f0 = lax.shift_right_logical(p0, 5)
    pltpu.sync_copy(xt_hbm.at[f0, :], x_v)

    def plane_body(pi, carry):
        p = p0 + pi
        f = lax.shift_right_logical(p, 5)
        d = lax.bitwise_and(p, 31)

        plane_cp = pltpu.async_copy(wt_hbm.at[f, d, :], plane_v, sem)

        @pl.when(jnp.logical_and(d == 0, pi > 0))
        def _new_field():
            pltpu.sync_copy(xt_hbm.at[f, :], x_v)

        plane_cp.wait()

        for q in range(4):
            slot = q % 2
            osem = osem0 if slot == 0 else osem1

            def _drain():
                pltpu.make_async_copy(
                    out_hbm.at[f, d, pl.ds(q * OUTC, OUTC)],
                    o_v.at[slot], osem,
                ).wait()

            if q >= 2:
                _drain()
            else:
                pl.when(pi > 0)(_drain)

            @plsc.parallel_loop(0, OUTC, 16, unroll=8)
            def _chunk_loop(i):
                xi = x_v[pl.ds(q * OUTC + i, 16)] + 1
                o_v[slot, pl.ds(i, 16)] = plsc.load_gather(plane_v, [xi])

            pltpu.async_copy(
                o_v.at[slot], out_hbm.at[f, d, pl.ds(q * OUTC, OUTC)], osem
            )
        return carry

    lax.fori_loop(0, PPW, plane_body, 0)

    pltpu.make_async_copy(
        out_hbm.at[0, 0, pl.ds(0, OUTC)], o_v.at[0], osem0).wait()
    pltpu.make_async_copy(
        out_hbm.at[0, 0, pl.ds(0, OUTC)], o_v.at[1], osem1).wait()


def kernel(x, W):
    wt = jnp.transpose(W, (0, 2, 1))
    xt = jnp.transpose(x, (1, 0))
    raw = _sc_gather(wt, xt)
    scale_full = _tc_scale(wt)
    out_t = _tc_tanh(raw, scale_full)
    return jnp.transpose(out_t, (2, 0, 1))
